# Initial kernel scaffold; baseline (speedup 1.0000x reference)
import jax
import jax.numpy as jnp
from jax.experimental import pallas as pl


def kernel(x, incidence, enc_w0, enc_b0, enc_ln_g, enc_ln_b, enc_w1, enc_b1, conv_w, dec_w0, dec_b0, dec_ln_g, dec_ln_b, dec_w1, dec_b1):
    raise NotImplementedError("write your pallas kernel here")


# speedup vs baseline: 1.8045x; 1.8045x over previous
import jax
import jax.numpy as jnp
from jax.experimental import pallas as pl
import jax.experimental.pallas.tpu as pltpu

N = 10000
IN_DIM = 128
HID = 64
OUT_DIM = 128

BM = 400
EPS = 1e-5


def _enc_body(x_ref, w0_ref, b0_ref, g_ref, b_ref, w1_ref, b1_ref, cw_ref,
              msg_ref):
    h = jnp.dot(x_ref[...], w0_ref[...], preferred_element_type=jnp.float32)
    h = jnp.maximum(h + b0_ref[...], 0.0)
    m = jnp.mean(h, axis=-1, keepdims=True)
    v = jnp.mean((h - m) ** 2, axis=-1, keepdims=True)
    h = (h - m) * jax.lax.rsqrt(v + EPS) * g_ref[...] + b_ref[...]
    h = jnp.dot(h, w1_ref[...], preferred_element_type=jnp.float32)
    h = jnp.maximum(h + b1_ref[...], 0.0)
    msg_ref[...] = jnp.dot(h, cw_ref[...], preferred_element_type=jnp.float32)


def _main_body(inc_ref, msg_ref, dw0_ref, db0_ref, g_ref, b_ref,
               dw1_ref, db1_ref, out_ref):
    inc = inc_ref[...]
    agg = jnp.dot(inc, msg_ref[...], preferred_element_type=jnp.float32)
    rs = jnp.sum(inc, axis=1, keepdims=True)
    agg = agg / rs
    d = jnp.dot(agg, dw0_ref[...], preferred_element_type=jnp.float32)
    d = jnp.maximum(d + db0_ref[...], 0.0)
    m = jnp.mean(d, axis=-1, keepdims=True)
    v = jnp.mean((d - m) ** 2, axis=-1, keepdims=True)
    d = (d - m) * jax.lax.rsqrt(v + EPS) * g_ref[...] + b_ref[...]
    d = jnp.dot(d, dw1_ref[...], preferred_element_type=jnp.float32)
    out_ref[...] = jnp.maximum(d + db1_ref[...], 0.0)


def kernel(x, incidence, enc_w0, enc_b0, enc_ln_g, enc_ln_b, enc_w1, enc_b1,
           conv_w, dec_w0, dec_b0, dec_ln_g, dec_ln_b, dec_w1, dec_b1):
    f32 = jnp.float32
    row2 = lambda a: a.reshape(1, -1)
    full = lambda shape: pl.BlockSpec(shape, lambda *_: (0,) * len(shape))

    nmb_enc = N // 1000
    msg = pl.pallas_call(
        _enc_body,
        grid=(nmb_enc,),
        in_specs=[
            pl.BlockSpec((1000, IN_DIM), lambda i: (i, 0)),
            full((IN_DIM, HID)),
            full((1, HID)),
            full((1, HID)),
            full((1, HID)),
            full((HID, HID)),
            full((1, HID)),
            full((HID, HID)),
        ],
        out_specs=pl.BlockSpec((1000, HID), lambda i: (i, 0)),
        out_shape=jax.ShapeDtypeStruct((N, HID), f32),
    )(x, enc_w0, row2(enc_b0), row2(enc_ln_g), row2(enc_ln_b), enc_w1,
      row2(enc_b1), conv_w)

    out = pl.pallas_call(
        _main_body,
        grid=(N // BM,),
        in_specs=[
            pl.BlockSpec((BM, N), lambda i: (i, 0)),
            full((N, HID)),
            full((HID, HID)),
            full((1, HID)),
            full((1, HID)),
            full((1, HID)),
            full((HID, OUT_DIM)),
            full((1, OUT_DIM)),
        ],
        out_specs=pl.BlockSpec((BM, OUT_DIM), lambda i: (i, 0)),
        out_shape=jax.ShapeDtypeStruct((N, OUT_DIM), f32),
        compiler_params=pltpu.CompilerParams(
            dimension_semantics=("arbitrary",),
        ),
    )(incidence, msg, dec_w0, row2(dec_b0), row2(dec_ln_g), row2(dec_ln_b),
      dec_w1, row2(dec_b1))
    return out


# speedup vs baseline: 1.8170x; 1.0069x over previous
import jax
import jax.numpy as jnp
from jax.experimental import pallas as pl
import jax.experimental.pallas.tpu as pltpu

N = 10000
IN_DIM = 128
HID = 64
OUT_DIM = 128

BM = 400
EPS = 1e-5


def _enc_body(x_ref, w0_ref, b0_ref, g_ref, b_ref, w1_ref, b1_ref, cw_ref,
              msg_ref):
    h = jnp.dot(x_ref[...], w0_ref[...], preferred_element_type=jnp.float32)
    h = jnp.maximum(h + b0_ref[...], 0.0)
    m = jnp.mean(h, axis=-1, keepdims=True)
    v = jnp.mean((h - m) ** 2, axis=-1, keepdims=True)
    h = (h - m) * jax.lax.rsqrt(v + EPS) * g_ref[...] + b_ref[...]
    h = jnp.dot(h, w1_ref[...], preferred_element_type=jnp.float32)
    h = jnp.maximum(h + b1_ref[...], 0.0)
    msgk = jnp.dot(h, cw_ref[...], preferred_element_type=jnp.float32)
    ones_col = (jax.lax.broadcasted_iota(jnp.int32, (msgk.shape[0], HID), 1)
                == 0).astype(jnp.float32)
    msg_ref[...] = jnp.concatenate([msgk, ones_col], axis=1)


def _main_body(inc_ref, msg_ref, dw0_ref, db0_ref, g_ref, b_ref,
               dw1_ref, db1_ref, out_ref):
    inc = inc_ref[...]
    res = jnp.dot(inc, msg_ref[...], preferred_element_type=jnp.float32)
    agg = res[:, :HID] / res[:, HID:HID + 1]
    d = jnp.dot(agg, dw0_ref[...], preferred_element_type=jnp.float32)
    d = jnp.maximum(d + db0_ref[...], 0.0)
    m = jnp.mean(d, axis=-1, keepdims=True)
    v = jnp.mean((d - m) ** 2, axis=-1, keepdims=True)
    d = (d - m) * jax.lax.rsqrt(v + EPS) * g_ref[...] + b_ref[...]
    d = jnp.dot(d, dw1_ref[...], preferred_element_type=jnp.float32)
    out_ref[...] = jnp.maximum(d + db1_ref[...], 0.0)


def kernel(x, incidence, enc_w0, enc_b0, enc_ln_g, enc_ln_b, enc_w1, enc_b1,
           conv_w, dec_w0, dec_b0, dec_ln_g, dec_ln_b, dec_w1, dec_b1):
    f32 = jnp.float32
    row2 = lambda a: a.reshape(1, -1)
    full = lambda shape: pl.BlockSpec(shape, lambda *_: (0,) * len(shape))

    nmb_enc = N // 1000
    msg = pl.pallas_call(
        _enc_body,
        grid=(nmb_enc,),
        in_specs=[
            pl.BlockSpec((1000, IN_DIM), lambda i: (i, 0)),
            full((IN_DIM, HID)),
            full((1, HID)),
            full((1, HID)),
            full((1, HID)),
            full((HID, HID)),
            full((1, HID)),
            full((HID, HID)),
        ],
        out_specs=pl.BlockSpec((1000, 2 * HID), lambda i: (i, 0)),
        out_shape=jax.ShapeDtypeStruct((N, 2 * HID), f32),
    )(x, enc_w0, row2(enc_b0), row2(enc_ln_g), row2(enc_ln_b), enc_w1,
      row2(enc_b1), conv_w)

    out = pl.pallas_call(
        _main_body,
        grid=(N // BM,),
        in_specs=[
            pl.BlockSpec((BM, N), lambda i: (i, 0)),
            full((N, 2 * HID)),
            full((HID, HID)),
            full((1, HID)),
            full((1, HID)),
            full((1, HID)),
            full((HID, OUT_DIM)),
            full((1, OUT_DIM)),
        ],
        out_specs=pl.BlockSpec((BM, OUT_DIM), lambda i: (i, 0)),
        out_shape=jax.ShapeDtypeStruct((N, OUT_DIM), f32),
        compiler_params=pltpu.CompilerParams(
            dimension_semantics=("arbitrary",),
        ),
    )(incidence, msg, dec_w0, row2(dec_b0), row2(dec_ln_g), row2(dec_ln_b),
      dec_w1, row2(dec_b1))
    return out
